# baseline (speedup 1.0000x reference)
import functools

import jax
import jax.numpy as jnp
from jax import lax
from jax.experimental import pallas as pl
from jax.experimental.pallas import tpu as pltpu
from jax.experimental.pallas import tpu_sc as plsc

N = 16384
C = 1000
NC = 2
NS = 16
L = 16
NW = NC * NS
NB = N // NW
NCHUNK = NB // 128


def _sc_body(prob_hbm, tgt_hbm, rwd_hbm, out_hbm, tgt_v, rwd_v, idx_v, gat_v,
             acc_v, sem):
    wid = lax.axis_index("s") * NC + lax.axis_index("c")
    base = wid * NB

    pltpu.sync_copy(tgt_hbm.at[pl.ds(base, NB)], tgt_v)
    pltpu.sync_copy(rwd_hbm.at[pl.ds(base, NB)], rwd_v)

    lane = lax.broadcasted_iota(jnp.int32, (L,), 0)
    for j in range(NB // L):
        t = tgt_v[pl.ds(j * L, L)]
        rows = (base + j * L) + lane
        idx_v[j // 8, pl.ds((j % 8) * L, L)] = rows * C + t

    copies = [
        pltpu.async_copy(prob_hbm.at[idx_v.at[k]],
                         gat_v.at[pl.ds(k * 128, 128)], sem)
        for k in range(NCHUNK)
    ]
    for cp in copies:
        cp.wait()

    acc = jnp.zeros((L,), jnp.float32)
    for j in range(NB // L):
        acc = acc + gat_v[pl.ds(j * L, L)] * rwd_v[pl.ds(j * L, L)]
    acc_v[...] = acc * (-1.0 / N)

    pltpu.sync_copy(acc_v, out_hbm.at[wid])


@jax.jit
def _ganloss(prob_flat, targets, reward):
    mesh = plsc.VectorSubcoreMesh(core_axis_name="c", subcore_axis_name="s")
    partials = pl.kernel(
        _sc_body,
        out_type=jax.ShapeDtypeStruct((NW, L), jnp.float32),
        mesh=mesh,
        scratch_types=[
            pltpu.VMEM((NB,), jnp.int32),
            pltpu.VMEM((NB,), jnp.float32),
            pltpu.VMEM((NCHUNK, 128), jnp.int32),
            pltpu.VMEM((NB,), jnp.float32),
            pltpu.VMEM((L,), jnp.float32),
            pltpu.SemaphoreType.DMA,
        ],
    )(prob_flat, targets, reward)
    return jnp.sum(partials)


def kernel(prob, targets, reward):
    return _ganloss(prob.reshape(-1), targets.astype(jnp.int32), reward)


# speedup vs baseline: 1.3285x; 1.3285x over previous
import functools

import jax
import jax.numpy as jnp
from jax import lax
from jax.experimental import pallas as pl
from jax.experimental.pallas import tpu as pltpu
from jax.experimental.pallas import tpu_sc as plsc

N = 16384
C = 1000
NC = 2
NS = 16
L = 16
NW = NC * NS
NB = N // NW
W = 128
NBKT = 8
CH = 32
PAD = NB + NBKT * CH


def _sc_body(prob_hbm, tgt_hbm, rwd_hbm, out_hbm, tgt_v, rwd_v, col_v, pos_v,
             idx_c, gat_v, acc_v, sem):
    wid = lax.axis_index("s") * NC + lax.axis_index("c")
    base = wid * NB

    pltpu.sync_copy(tgt_hbm.at[pl.ds(base, NB)], tgt_v)
    pltpu.sync_copy(rwd_hbm.at[pl.ds(base, NB)], rwd_v)

    lane = lax.broadcasted_iota(jnp.int32, (L,), 0)
    zeros = jnp.zeros((L,), jnp.int32)

    def zinit(j, carry):
        idx_c[pl.ds(j * L, L)] = zeros
        return carry

    lax.fori_loop(0, PAD // L, zinit, 0, unroll=False)

    def colprep(j, carry):
        t = tgt_v[pl.ds(j * L, L)]
        col_v[pl.ds(j * L, L)] = lax.bitwise_and(t, W - 1)
        return carry

    lax.fori_loop(0, NB // L, colprep, 0, unroll=False)

    cursor = jnp.int32(0)
    starts = []
    counts = []
    for b in range(NBKT):
        starts.append(cursor)

        def compact(j, cur, b=b):
            t = tgt_v[pl.ds(j * L, L)]
            mask = lax.shift_right_logical(t, 7) == b
            rows = (base + j * L) + lane
            rank = plsc.cumsum(jnp.where(mask, 1, 0))
            newpos = cur + rank - 1
            plsc.store_scatter(idx_c, [newpos], rows, mask=mask)
            old = pos_v[pl.ds(j * L, L)]
            pos_v[pl.ds(j * L, L)] = jnp.where(mask, newpos, old)
            npop = plsc.all_reduce_population_count(mask)
            return cur + npop[0]

        cursor = lax.fori_loop(0, NB // L, compact, cursor, unroll=False)
        counts.append(cursor - starts[b])
        cursor = lax.bitwise_and(cursor + (CH - 1), ~jnp.int32(CH - 1))

    total_chunks = jnp.int32(0)
    for b in range(NBKT):
        nchunks = lax.shift_right_logical(counts[b] + (CH - 1), 5)
        total_chunks = total_chunks + nchunks
        cstart = pl.multiple_of(jnp.full((), b * W, jnp.int32), W)

        def fire(k, carry, b=b, nchunks=nchunks, cstart=cstart):
            dst0 = pl.multiple_of(starts[b] + k * CH, CH)
            pltpu.async_copy(
                prob_hbm.at[plsc.Indices(idx_c.at[pl.ds(dst0, CH)]),
                            pl.ds(cstart, W)],
                gat_v.at[pl.ds(dst0, CH)], sem)
            return carry

        lax.fori_loop(0, nchunks, fire, 0, unroll=False)

    def drain(i, carry):
        pltpu.make_async_copy(
            prob_hbm.at[plsc.Indices(idx_c.at[pl.ds(0, CH)]), pl.ds(0, W)],
            gat_v.at[pl.ds(0, CH)], sem).wait()
        return carry

    lax.fori_loop(0, total_chunks, drain, 0, unroll=False)

    def accum(j, acc):
        pos = pos_v[pl.ds(j * L, L)]
        cols = col_v[pl.ds(j * L, L)]
        vals = plsc.load_gather(gat_v, [pos, cols])
        return acc + vals * rwd_v[pl.ds(j * L, L)]

    acc = lax.fori_loop(0, NB // L, accum, jnp.zeros((L,), jnp.float32),
                        unroll=False)
    acc_v[...] = acc * (-1.0 / N)

    pltpu.sync_copy(acc_v, out_hbm.at[wid])


@jax.jit
def _ganloss(prob, targets, reward):
    mesh = plsc.VectorSubcoreMesh(core_axis_name="c", subcore_axis_name="s")
    partials = pl.kernel(
        _sc_body,
        out_type=jax.ShapeDtypeStruct((NW, L), jnp.float32),
        mesh=mesh,
        compiler_params=pltpu.CompilerParams(needs_layout_passes=False),
        scratch_types=[
            pltpu.VMEM((NB,), jnp.int32),
            pltpu.VMEM((NB,), jnp.float32),
            pltpu.VMEM((NB,), jnp.int32),
            pltpu.VMEM((NB,), jnp.int32),
            pltpu.VMEM((PAD,), jnp.int32),
            pltpu.VMEM((PAD, W), jnp.float32),
            pltpu.VMEM((L,), jnp.float32),
            pltpu.SemaphoreType.DMA,
        ],
    )(prob, targets, reward)
    return jnp.sum(partials)


def kernel(prob, targets, reward):
    return _ganloss(prob, targets.astype(jnp.int32), reward)


# speedup vs baseline: 5.2489x; 3.9510x over previous
import functools

import jax
import jax.numpy as jnp
from jax import lax
from jax.experimental import pallas as pl
from jax.experimental.pallas import tpu as pltpu
from jax.experimental.pallas import tpu_sc as plsc

N = 16384
C = 1000
NC = 2
NS = 16
L = 16
NW = NC * NS
NB = N // NW
W = 128
NQ = NB // W


def _sc_body(probT_hbm, tgt_hbm, rwd_hbm, out_hbm, tgt_v, rwd_v, gat_v,
             acc_v, sem):
    wid = lax.axis_index("s") * NC + lax.axis_index("c")
    base = wid * NB

    pltpu.sync_copy(tgt_hbm.at[pl.ds(base, NB)], tgt_v)
    pltpu.sync_copy(rwd_hbm.at[pl.ds(base, NB)], rwd_v)

    copies = []
    for q in range(NQ):
        cstart = pl.multiple_of(base + q * W, W)
        copies.append(pltpu.async_copy(
            probT_hbm.at[plsc.Indices(tgt_v.at[pl.ds(q * W, W)]),
                         pl.ds(cstart, W)],
            gat_v.at[pl.ds(q * W, W)], sem))
    for cp in copies:
        cp.wait()

    lane = lax.broadcasted_iota(jnp.int32, (L,), 0)

    def accum(j, acc):
        rows = j * L + lane
        cols = lax.bitwise_and(rows, W - 1)
        vals = plsc.load_gather(gat_v, [rows, cols])
        return acc + vals * rwd_v[pl.ds(j * L, L)]

    acc = lax.fori_loop(0, NB // L, accum, jnp.zeros((L,), jnp.float32),
                        unroll=False)
    acc_v[...] = acc * (-1.0 / N)

    pltpu.sync_copy(acc_v, out_hbm.at[wid])


@jax.jit
def _ganloss(prob, targets, reward):
    mesh = plsc.VectorSubcoreMesh(core_axis_name="c", subcore_axis_name="s")
    partials = pl.kernel(
        _sc_body,
        out_type=jax.ShapeDtypeStruct((NW, L), jnp.float32),
        mesh=mesh,
        compiler_params=pltpu.CompilerParams(needs_layout_passes=False),
        scratch_types=[
            pltpu.VMEM((NB,), jnp.int32),
            pltpu.VMEM((NB,), jnp.float32),
            pltpu.VMEM((NB, W), jnp.float32),
            pltpu.VMEM((L,), jnp.float32),
            pltpu.SemaphoreType.DMA,
        ],
    )(jnp.swapaxes(prob, 0, 1), targets, reward)
    return jnp.sum(partials)


def kernel(prob, targets, reward):
    return _ganloss(prob, targets.astype(jnp.int32), reward)
